# Initial kernel scaffold; baseline (speedup 1.0000x reference)
import jax
import jax.numpy as jnp
from jax.experimental import pallas as pl


def kernel(batch_input, adj, weight, bias):
    raise NotImplementedError("write your pallas kernel here")


# speedup vs baseline: 1.8126x; 1.8126x over previous
import jax
import jax.numpy as jnp
from jax.experimental import pallas as pl
from jax.experimental.pallas import tpu as pltpu


def _support_body(x_ref, w_ref, s_ref):
    w = w_ref[...]
    for b in range(x_ref.shape[0]):
        s_ref[b] = jnp.dot(x_ref[b], w, preferred_element_type=jnp.float32)


def _spmm_body(adj_ref, s_ref, bias_ref, o_ref):
    a = adj_ref[...]
    bias = bias_ref[...]
    for b in range(s_ref.shape[0]):
        o_ref[b] = jnp.dot(a, s_ref[b],
                           preferred_element_type=jnp.float32) + bias


def kernel(batch_input, adj, weight, bias):
    if batch_input.ndim == 2:
        batch_input = batch_input[None]
    nb, n, f_in = batch_input.shape
    f_out = weight.shape[1]
    m = adj.shape[0]

    bm_s = 2000 if n % 2000 == 0 else n
    support = pl.pallas_call(
        _support_body,
        grid=(n // bm_s,),
        in_specs=[
            pl.BlockSpec((nb, bm_s, f_in), lambda i: (0, i, 0)),
            pl.BlockSpec((f_in, f_out), lambda i: (0, 0)),
        ],
        out_specs=pl.BlockSpec((nb, bm_s, f_out), lambda i: (0, i, 0)),
        out_shape=jax.ShapeDtypeStruct((nb, n, f_out), jnp.float32),
        compiler_params=pltpu.CompilerParams(
            dimension_semantics=("parallel",),
        ),
    )(batch_input, weight)

    bm = 400 if m % 400 == 0 else m
    bias2d = bias.reshape(1, f_out)

    out = pl.pallas_call(
        _spmm_body,
        grid=(m // bm,),
        in_specs=[
            pl.BlockSpec((bm, n), lambda i: (i, 0)),
            pl.BlockSpec((nb, n, f_out), lambda i: (0, 0, 0)),
            pl.BlockSpec((1, f_out), lambda i: (0, 0)),
        ],
        out_specs=pl.BlockSpec((nb, bm, f_out), lambda i: (0, i, 0)),
        out_shape=jax.ShapeDtypeStruct((nb, m, f_out), jnp.float32),
        compiler_params=pltpu.CompilerParams(
            dimension_semantics=("parallel",),
        ),
    )(adj, support, bias2d)

    return out


# speedup vs baseline: 1.9261x; 1.0626x over previous
import jax
import jax.numpy as jnp
from jax.experimental import pallas as pl
from jax.experimental.pallas import tpu as pltpu


def _fused_body(x_ref, w_ref, adj_ref, bias_ref, o_ref, s_ref):
    i = pl.program_id(0)

    @pl.when(i == 0)
    def _compute_support():
        w = w_ref[...]
        for b in range(x_ref.shape[0]):
            s_ref[b] = jnp.dot(x_ref[b], w, preferred_element_type=jnp.float32)

    a = adj_ref[...]
    bias = bias_ref[...]
    for b in range(s_ref.shape[0]):
        o_ref[b] = jnp.dot(a, s_ref[b],
                           preferred_element_type=jnp.float32) + bias


def kernel(batch_input, adj, weight, bias):
    if batch_input.ndim == 2:
        batch_input = batch_input[None]
    nb, n, f_in = batch_input.shape
    f_out = weight.shape[1]
    m = adj.shape[0]

    bm = 400 if m % 400 == 0 else m
    bias2d = bias.reshape(1, f_out)

    out = pl.pallas_call(
        _fused_body,
        grid=(m // bm,),
        in_specs=[
            pl.BlockSpec((nb, n, f_in), lambda i: (0, 0, 0)),
            pl.BlockSpec((f_in, f_out), lambda i: (0, 0)),
            pl.BlockSpec((bm, n), lambda i: (i, 0)),
            pl.BlockSpec((1, f_out), lambda i: (0, 0)),
        ],
        out_specs=pl.BlockSpec((nb, bm, f_out), lambda i: (0, i, 0)),
        out_shape=jax.ShapeDtypeStruct((nb, m, f_out), jnp.float32),
        scratch_shapes=[pltpu.VMEM((nb, n, f_out), jnp.float32)],
        compiler_params=pltpu.CompilerParams(
            dimension_semantics=("arbitrary",),
        ),
    )(batch_input, weight, adj, bias2d)

    return out
